# baseline (speedup 1.0000x reference)
import functools

import jax
import jax.numpy as jnp
from jax import lax
from jax.experimental import pallas as pl
from jax.experimental.pallas import tpu as pltpu
from jax.experimental.pallas import tpu_sc as plsc

L = 16
NC = 2
NS = 16
NW = NC * NS


def _build_transpose(D=96, H=384, W=384, hb=8):
    def tbody(fm_ref, out_ref):
        x = fm_ref[...]
        x2 = x.reshape(D, hb * W)
        out_ref[...] = jnp.transpose(x2, (1, 0)).astype(jnp.bfloat16)

    return pl.pallas_call(
        tbody,
        grid=(H // hb,),
        in_specs=[pl.BlockSpec((D, hb, W), lambda h: (0, h, 0))],
        out_specs=pl.BlockSpec((hb * W, D), lambda h: (h, 0)),
        out_shape=jax.ShapeDtypeStruct((H * W, D), jnp.bfloat16),
    )


def _build_sampler(D=96, H=384, W=384, N=16384, chunk=64):
    per_tile = N // NW
    assert per_tile % chunk == 0
    nchunk = per_tile // chunk
    ngrp = per_tile // L
    assert chunk <= 128
    assert D % (2 * L) == 0
    assert nchunk % 2 == 0

    def body(table, pts, out, pts_v,
             idx00, idx10, idx01, idx11,
             w00_v, w10_v, w01_v, w11_v,
             rows0, rows1, out_v, sem):
        cid = lax.axis_index("c")
        sid = lax.axis_index("s")
        wid = sid * NC + cid
        base = wid * per_tile

        pltpu.sync_copy(pts.at[pl.ds(base * 2, per_tile * 2)], pts_v)

        lanes2 = lax.iota(jnp.int32, L) * 2

        def grp(g, carry):
            sl = pl.ds(g * L, L)
            ii = lanes2 + g * 2 * L
            x = plsc.load_gather(pts_v, [ii])
            y = plsc.load_gather(pts_v, [ii + 1])
            ix = x - 0.5
            iy = y - 0.5
            x0 = ix.astype(jnp.int32)
            x0 = jnp.where(ix < x0.astype(jnp.float32), x0 - 1, x0)
            y0 = iy.astype(jnp.int32)
            y0 = jnp.where(iy < y0.astype(jnp.float32), y0 - 1, y0)
            wx1 = ix - x0.astype(jnp.float32)
            wx0 = 1.0 - wx1
            wy1 = iy - y0.astype(jnp.float32)
            wy0 = 1.0 - wy1
            wx0 = jnp.where(x0 >= 0, wx0, 0.0)
            wx1 = jnp.where(x0 <= W - 2, wx1, 0.0)
            wy0 = jnp.where(y0 >= 0, wy0, 0.0)
            wy1 = jnp.where(y0 <= H - 2, wy1, 0.0)
            x0c = jnp.maximum(x0, 0)
            x1c = jnp.minimum(x0 + 1, W - 1)
            y0c = jnp.maximum(y0, 0)
            y1c = jnp.minimum(y0 + 1, H - 1)
            r0 = y0c * W
            r1 = y1c * W
            c = g // (chunk // L)
            o = (g % (chunk // L)) * L
            csl = pl.ds(o, L)
            idx00[c, csl] = r0 + x0c
            idx10[c, csl] = r0 + x1c
            idx01[c, csl] = r1 + x0c
            idx11[c, csl] = r1 + x1c
            w00_v[sl] = wx0 * wy0
            w10_v[sl] = wx1 * wy0
            w01_v[sl] = wx0 * wy1
            w11_v[sl] = wx1 * wy1
            return carry

        lax.fori_loop(0, ngrp, grp, 0)

        def fire(c, buf):
            pltpu.async_copy(table.at[idx00.at[c]], buf.at[0], sem)
            pltpu.async_copy(table.at[idx10.at[c]], buf.at[1], sem)
            pltpu.async_copy(table.at[idx01.at[c]], buf.at[2], sem)
            pltpu.async_copy(table.at[idx11.at[c]], buf.at[3], sem)

        def drain(c, buf):
            pltpu.make_async_copy(table.at[idx00.at[c]], buf.at[0], sem).wait()
            pltpu.make_async_copy(table.at[idx10.at[c]], buf.at[1], sem).wait()
            pltpu.make_async_copy(table.at[idx01.at[c]], buf.at[2], sem).wait()
            pltpu.make_async_copy(table.at[idx11.at[c]], buf.at[3], sem).wait()

        hi_mask = jnp.full((L,), -65536, dtype=jnp.int32)

        def halves(v32):
            xi = plsc.bitcast(v32, jnp.int32)
            lo = plsc.bitcast(lax.shift_left(xi, 16), jnp.float32)
            hi = plsc.bitcast(lax.bitwise_and(xi, hi_mask), jnp.float32)
            return lo, hi

        def combine(c, buf):
            cbase = c * chunk

            def grp16(g, carry2):
                gb = g * L
                w00v = w00_v[pl.ds(cbase + gb, L)]
                w10v = w10_v[pl.ds(cbase + gb, L)]
                w01v = w01_v[pl.ds(cbase + gb, L)]
                w11v = w11_v[pl.ds(cbase + gb, L)]
                for k in range(L):
                    p = gb + k
                    w00 = w00v[k]
                    w10 = w10v[k]
                    w01 = w01v[k]
                    w11 = w11v[k]
                    for j in range(D // (2 * L)):
                        cs = pl.ds(j * 2 * L, 2 * L)
                        lo0, hi0 = halves(buf[0, p, cs])
                        lo1, hi1 = halves(buf[1, p, cs])
                        lo2, hi2 = halves(buf[2, p, cs])
                        lo3, hi3 = halves(buf[3, p, cs])
                        even = (w00 * lo0 + w10 * lo1 + w01 * lo2 + w11 * lo3)
                        odd = (w00 * hi0 + w10 * hi1 + w01 * hi2 + w11 * hi3)
                        out_v[p, cs] = plsc.pack(
                            even, odd, format=plsc.PackFormat.INTERLEAVED)
                return carry2

            lax.fori_loop(0, chunk // L, grp16, 0)
            pltpu.sync_copy(out_v, out.at[pl.ds(base + cbase, chunk)])

        fire(0, rows0)

        def pair(i2, carry):
            c0 = i2 * 2
            drain(c0, rows0)
            fire(c0 + 1, rows1)
            combine(c0, rows0)
            drain(c0 + 1, rows1)

            @pl.when(c0 + 2 < nchunk)
            def _():
                fire(c0 + 2, rows0)

            combine(c0 + 1, rows1)
            return carry

        lax.fori_loop(0, nchunk // 2, pair, 0)

    mesh = plsc.VectorSubcoreMesh(core_axis_name="c", subcore_axis_name="s",
                                  num_cores=NC, num_subcores=NS)
    return pl.kernel(
        body,
        out_type=jax.ShapeDtypeStruct((N, D), jnp.bfloat16),
        mesh=mesh,
        scratch_types=[
            pltpu.VMEM((2 * per_tile,), jnp.float32),
            pltpu.VMEM((nchunk, chunk), jnp.int32),
            pltpu.VMEM((nchunk, chunk), jnp.int32),
            pltpu.VMEM((nchunk, chunk), jnp.int32),
            pltpu.VMEM((nchunk, chunk), jnp.int32),
            pltpu.VMEM((per_tile + L,), jnp.float32),
            pltpu.VMEM((per_tile + L,), jnp.float32),
            pltpu.VMEM((per_tile + L,), jnp.float32),
            pltpu.VMEM((per_tile + L,), jnp.float32),
            pltpu.VMEM((4, chunk, D), jnp.bfloat16),
            pltpu.VMEM((4, chunk, D), jnp.bfloat16),
            pltpu.VMEM((chunk, D), jnp.bfloat16),
            pltpu.SemaphoreType.DMA,
        ],
        compiler_params=pltpu.CompilerParams(use_tc_tiling_on_sc=False,
                                             needs_layout_passes=False),
    )


_transpose = _build_transpose()
_sampler = _build_sampler()


@jax.jit
def kernel(feature_maps, sample_points):
    B, D, H, W = feature_maps.shape
    N = sample_points.shape[1]
    outs = []
    for b in range(B):
        table = _transpose(feature_maps[b])
        pts = sample_points[b].reshape(2 * N)
        outs.append(_sampler(table, pts))
    out = jnp.stack(outs)
    return out.astype(jnp.float32)


# speedup vs baseline: 1.1288x; 1.1288x over previous
import functools

import jax
import jax.numpy as jnp
from jax import lax
from jax.experimental import pallas as pl
from jax.experimental.pallas import tpu as pltpu
from jax.experimental.pallas import tpu_sc as plsc

L = 16
NC = 2
NS = 16
NW = NC * NS


def _build_transpose(B=2, D=96, H=384, W=384, hb=8):
    def tbody(fm_ref, out_ref):
        x = fm_ref[0]
        x2 = x.reshape(D, hb * W)
        eye = (lax.broadcasted_iota(jnp.int32, (D, D), 0)
               == lax.broadcasted_iota(jnp.int32, (D, D), 1)).astype(jnp.float32)
        dn = (((0,), (0,)), ((), ()))
        y = lax.dot_general(x2, eye, dn, preferred_element_type=jnp.float32)
        out_ref[...] = y.astype(jnp.bfloat16)

    return pl.pallas_call(
        tbody,
        grid=(B, H // hb),
        in_specs=[pl.BlockSpec((1, D, hb, W), lambda b, h: (b, 0, h, 0))],
        out_specs=pl.BlockSpec((hb * W, D), lambda b, h: (b * (H // hb) + h, 0)),
        out_shape=jax.ShapeDtypeStruct((B * H * W, D), jnp.bfloat16),
    )


def _build_sampler(B=2, D=96, H=384, W=384, N=16384, chunk=64):
    total = B * N
    per_tile = total // NW
    assert per_tile % chunk == 0
    nchunk = per_tile // chunk
    ngrp = per_tile // L
    assert N % per_tile == 0
    assert chunk <= 128
    assert D % (2 * L) == 0
    assert nchunk % 2 == 0

    def body(table, pts, out, pts_v,
             idx00, idx10, idx01, idx11,
             w00_v, w10_v, w01_v, w11_v,
             rows0, rows1, out_v, sem):
        cid = lax.axis_index("c")
        sid = lax.axis_index("s")
        wid = sid * NC + cid
        base = wid * per_tile
        row_base = (base // N) * (H * W)

        pltpu.sync_copy(pts.at[pl.ds(base * 2, per_tile * 2)], pts_v)

        lanes2 = lax.iota(jnp.int32, L) * 2

        def grp(g, carry):
            sl = pl.ds(g * L, L)
            ii = lanes2 + g * 2 * L
            x = plsc.load_gather(pts_v, [ii])
            y = plsc.load_gather(pts_v, [ii + 1])
            ix = x - 0.5
            iy = y - 0.5
            x0 = ix.astype(jnp.int32)
            x0 = jnp.where(ix < x0.astype(jnp.float32), x0 - 1, x0)
            y0 = iy.astype(jnp.int32)
            y0 = jnp.where(iy < y0.astype(jnp.float32), y0 - 1, y0)
            wx1 = ix - x0.astype(jnp.float32)
            wx0 = 1.0 - wx1
            wy1 = iy - y0.astype(jnp.float32)
            wy0 = 1.0 - wy1
            wx0 = jnp.where(x0 >= 0, wx0, 0.0)
            wx1 = jnp.where(x0 <= W - 2, wx1, 0.0)
            wy0 = jnp.where(y0 >= 0, wy0, 0.0)
            wy1 = jnp.where(y0 <= H - 2, wy1, 0.0)
            x0c = jnp.maximum(x0, 0)
            x1c = jnp.minimum(x0 + 1, W - 1)
            y0c = jnp.maximum(y0, 0)
            y1c = jnp.minimum(y0 + 1, H - 1)
            r0 = row_base + y0c * W
            r1 = row_base + y1c * W
            c = g // (chunk // L)
            o = (g % (chunk // L)) * L
            csl = pl.ds(o, L)
            idx00[c, csl] = r0 + x0c
            idx10[c, csl] = r0 + x1c
            idx01[c, csl] = r1 + x0c
            idx11[c, csl] = r1 + x1c
            w00_v[sl] = wx0 * wy0
            w10_v[sl] = wx1 * wy0
            w01_v[sl] = wx0 * wy1
            w11_v[sl] = wx1 * wy1
            return carry

        lax.fori_loop(0, ngrp, grp, 0)

        def fire(c, buf):
            pltpu.async_copy(table.at[idx00.at[c]], buf.at[0], sem)
            pltpu.async_copy(table.at[idx10.at[c]], buf.at[1], sem)
            pltpu.async_copy(table.at[idx01.at[c]], buf.at[2], sem)
            pltpu.async_copy(table.at[idx11.at[c]], buf.at[3], sem)

        def drain(c, buf):
            pltpu.make_async_copy(table.at[idx00.at[c]], buf.at[0], sem).wait()
            pltpu.make_async_copy(table.at[idx10.at[c]], buf.at[1], sem).wait()
            pltpu.make_async_copy(table.at[idx01.at[c]], buf.at[2], sem).wait()
            pltpu.make_async_copy(table.at[idx11.at[c]], buf.at[3], sem).wait()

        hi_mask = jnp.full((L,), -65536, dtype=jnp.int32)
        lanes_even = lax.iota(jnp.int32, L) * 2

        def halves(v32):
            xi = plsc.bitcast(v32, jnp.int32)
            lo = plsc.bitcast(lax.shift_left(xi, 16), jnp.float32)
            hi = plsc.bitcast(lax.bitwise_and(xi, hi_mask), jnp.float32)
            return lo, hi

        def combine(c, buf):
            cbase = c * chunk

            def grp16(g, carry2):
                gb = g * L
                w00v = w00_v[pl.ds(cbase + gb, L)]
                w10v = w10_v[pl.ds(cbase + gb, L)]
                w01v = w01_v[pl.ds(cbase + gb, L)]
                w11v = w11_v[pl.ds(cbase + gb, L)]
                for k in range(L):
                    p = gb + k
                    w00 = w00v[k]
                    w10 = w10v[k]
                    w01 = w01v[k]
                    w11 = w11v[k]
                    pD = p * D
                    for j in range(D // (2 * L)):
                        cs = pl.ds(j * 2 * L, 2 * L)
                        lo0, hi0 = halves(buf[0, p, cs])
                        lo1, hi1 = halves(buf[1, p, cs])
                        lo2, hi2 = halves(buf[2, p, cs])
                        lo3, hi3 = halves(buf[3, p, cs])
                        even = (w00 * lo0 + w10 * lo1 + w01 * lo2 + w11 * lo3)
                        odd = (w00 * hi0 + w10 * hi1 + w01 * hi2 + w11 * hi3)
                        ei = lanes_even + (pD + j * 2 * L)
                        plsc.store_scatter(out_v, [ei], even)
                        plsc.store_scatter(out_v, [ei + 1], odd)
                return carry2

            lax.fori_loop(0, chunk // L, grp16, 0)
            pltpu.sync_copy(out_v, out.at[pl.ds((base + cbase) * D, chunk * D)])

        fire(0, rows0)

        def pair(i2, carry):
            c0 = i2 * 2
            drain(c0, rows0)
            fire(c0 + 1, rows1)
            combine(c0, rows0)
            drain(c0 + 1, rows1)

            @pl.when(c0 + 2 < nchunk)
            def _():
                fire(c0 + 2, rows0)

            combine(c0 + 1, rows1)
            return carry

        lax.fori_loop(0, nchunk // 2, pair, 0)

    mesh = plsc.VectorSubcoreMesh(core_axis_name="c", subcore_axis_name="s",
                                  num_cores=NC, num_subcores=NS)
    return pl.kernel(
        body,
        out_type=jax.ShapeDtypeStruct((total * D,), jnp.float32),
        mesh=mesh,
        scratch_types=[
            pltpu.VMEM((2 * per_tile,), jnp.float32),
            pltpu.VMEM((nchunk, chunk), jnp.int32),
            pltpu.VMEM((nchunk, chunk), jnp.int32),
            pltpu.VMEM((nchunk, chunk), jnp.int32),
            pltpu.VMEM((nchunk, chunk), jnp.int32),
            pltpu.VMEM((per_tile + L,), jnp.float32),
            pltpu.VMEM((per_tile + L,), jnp.float32),
            pltpu.VMEM((per_tile + L,), jnp.float32),
            pltpu.VMEM((per_tile + L,), jnp.float32),
            pltpu.VMEM((4, chunk, D), jnp.bfloat16),
            pltpu.VMEM((4, chunk, D), jnp.bfloat16),
            pltpu.VMEM((chunk * D,), jnp.float32),
            pltpu.SemaphoreType.DMA,
        ],
        compiler_params=pltpu.CompilerParams(use_tc_tiling_on_sc=False,
                                             needs_layout_passes=False),
    )


_transpose = _build_transpose()
_sampler = _build_sampler()


@jax.jit
def kernel(feature_maps, sample_points):
    B, D, H, W = feature_maps.shape
    N = sample_points.shape[1]
    table = _transpose(feature_maps)
    pts = sample_points.reshape(2 * B * N)
    out = _sampler(table, pts)
    return out.reshape(B, N, D)


# speedup vs baseline: 1.9622x; 1.7384x over previous
import functools

import jax
import jax.numpy as jnp
from jax import lax
from jax.experimental import pallas as pl
from jax.experimental.pallas import tpu as pltpu
from jax.experimental.pallas import tpu_sc as plsc

L = 16
NC = 2
NS = 16
NW = NC * NS
DP = 128


def _build_transpose(B=2, D=96, H=384, W=384, hb=8):
    def tbody(fm_ref, out_ref):
        x = fm_ref[0]
        x2 = x.reshape(D, hb * W)
        eye = (lax.broadcasted_iota(jnp.int32, (D, DP), 0)
               == lax.broadcasted_iota(jnp.int32, (D, DP), 1)).astype(jnp.float32)
        dn = (((0,), (0,)), ((), ()))
        out_ref[...] = lax.dot_general(x2, eye, dn,
                                       preferred_element_type=jnp.float32)

    return pl.pallas_call(
        tbody,
        grid=(B, H // hb),
        in_specs=[pl.BlockSpec((1, D, hb, W), lambda b, h: (b, 0, h, 0))],
        out_specs=pl.BlockSpec((hb * W, DP), lambda b, h: (b * (H // hb) + h, 0)),
        out_shape=jax.ShapeDtypeStruct((B * H * W, DP), jnp.float32),
    )


def _build_sampler(B=2, D=96, H=384, W=384, N=16384, chunk=64):
    total = B * N
    per_tile = total // NW
    assert per_tile % chunk == 0
    nchunk = per_tile // chunk
    ngrp = per_tile // L
    assert N % per_tile == 0
    assert chunk <= 128
    assert D % L == 0
    assert nchunk % 2 == 0

    def body(table, pts, out, pts_v,
             idx00, idx10, idx01, idx11,
             w00_v, w10_v, w01_v, w11_v,
             rows0, rows1, out_v, sem):
        cid = lax.axis_index("c")
        sid = lax.axis_index("s")
        wid = sid * NC + cid
        base = wid * per_tile
        row_base = (base // N) * (H * W)

        pltpu.sync_copy(pts.at[pl.ds(base * 2, per_tile * 2)], pts_v)

        lanes2 = lax.iota(jnp.int32, L) * 2

        def grp(g, carry):
            sl = pl.ds(g * L, L)
            ii = lanes2 + g * 2 * L
            x = plsc.load_gather(pts_v, [ii])
            y = plsc.load_gather(pts_v, [ii + 1])
            ix = x - 0.5
            iy = y - 0.5
            x0 = ix.astype(jnp.int32)
            x0 = jnp.where(ix < x0.astype(jnp.float32), x0 - 1, x0)
            y0 = iy.astype(jnp.int32)
            y0 = jnp.where(iy < y0.astype(jnp.float32), y0 - 1, y0)
            wx1 = ix - x0.astype(jnp.float32)
            wx0 = 1.0 - wx1
            wy1 = iy - y0.astype(jnp.float32)
            wy0 = 1.0 - wy1
            wx0 = jnp.where(x0 >= 0, wx0, 0.0)
            wx1 = jnp.where(x0 <= W - 2, wx1, 0.0)
            wy0 = jnp.where(y0 >= 0, wy0, 0.0)
            wy1 = jnp.where(y0 <= H - 2, wy1, 0.0)
            x0c = jnp.maximum(x0, 0)
            x1c = jnp.minimum(x0 + 1, W - 1)
            y0c = jnp.maximum(y0, 0)
            y1c = jnp.minimum(y0 + 1, H - 1)
            r0 = row_base + y0c * W
            r1 = row_base + y1c * W
            c = g // (chunk // L)
            o = (g % (chunk // L)) * L
            csl = pl.ds(o, L)
            idx00[c, csl] = r0 + x0c
            idx10[c, csl] = r0 + x1c
            idx01[c, csl] = r1 + x0c
            idx11[c, csl] = r1 + x1c
            w00_v[sl] = wx0 * wy0
            w10_v[sl] = wx1 * wy0
            w01_v[sl] = wx0 * wy1
            w11_v[sl] = wx1 * wy1
            return carry

        lax.fori_loop(0, ngrp, grp, 0)

        def fire(c, buf):
            pltpu.async_copy(table.at[idx00.at[c]], buf.at[0], sem)
            pltpu.async_copy(table.at[idx10.at[c]], buf.at[1], sem)
            pltpu.async_copy(table.at[idx01.at[c]], buf.at[2], sem)
            pltpu.async_copy(table.at[idx11.at[c]], buf.at[3], sem)

        def drain(c, buf):
            pltpu.make_async_copy(table.at[idx00.at[c]], buf.at[0], sem).wait()
            pltpu.make_async_copy(table.at[idx10.at[c]], buf.at[1], sem).wait()
            pltpu.make_async_copy(table.at[idx01.at[c]], buf.at[2], sem).wait()
            pltpu.make_async_copy(table.at[idx11.at[c]], buf.at[3], sem).wait()

        def combine(c, buf):
            cbase = c * chunk

            def grp16(g, carry2):
                gb = g * L
                w00v = w00_v[pl.ds(cbase + gb, L)]
                w10v = w10_v[pl.ds(cbase + gb, L)]
                w01v = w01_v[pl.ds(cbase + gb, L)]
                w11v = w11_v[pl.ds(cbase + gb, L)]
                for k in range(L):
                    p = gb + k
                    w00 = w00v[k]
                    w10 = w10v[k]
                    w01 = w01v[k]
                    w11 = w11v[k]
                    pD = p * D
                    for j in range(D // L):
                        cs = pl.ds(j * L, L)
                        acc = (w00 * buf[0, p, cs] + w10 * buf[1, p, cs]
                               + w01 * buf[2, p, cs] + w11 * buf[3, p, cs])
                        out_v[pl.ds(pD + j * L, L)] = acc
                return carry2

            lax.fori_loop(0, chunk // L, grp16, 0)
            pltpu.sync_copy(out_v, out.at[pl.ds((base + cbase) * D, chunk * D)])

        fire(0, rows0)

        def pair(i2, carry):
            c0 = i2 * 2
            drain(c0, rows0)
            fire(c0 + 1, rows1)
            combine(c0, rows0)
            drain(c0 + 1, rows1)

            @pl.when(c0 + 2 < nchunk)
            def _():
                fire(c0 + 2, rows0)

            combine(c0 + 1, rows1)
            return carry

        lax.fori_loop(0, nchunk // 2, pair, 0)

    mesh = plsc.VectorSubcoreMesh(core_axis_name="c", subcore_axis_name="s",
                                  num_cores=NC, num_subcores=NS)
    return pl.kernel(
        body,
        out_type=jax.ShapeDtypeStruct((total * D,), jnp.float32),
        mesh=mesh,
        scratch_types=[
            pltpu.VMEM((2 * per_tile,), jnp.float32),
            pltpu.VMEM((nchunk, chunk), jnp.int32),
            pltpu.VMEM((nchunk, chunk), jnp.int32),
            pltpu.VMEM((nchunk, chunk), jnp.int32),
            pltpu.VMEM((nchunk, chunk), jnp.int32),
            pltpu.VMEM((per_tile + L,), jnp.float32),
            pltpu.VMEM((per_tile + L,), jnp.float32),
            pltpu.VMEM((per_tile + L,), jnp.float32),
            pltpu.VMEM((per_tile + L,), jnp.float32),
            pltpu.VMEM((4, chunk, DP), jnp.float32),
            pltpu.VMEM((4, chunk, DP), jnp.float32),
            pltpu.VMEM((chunk * D,), jnp.float32),
            pltpu.SemaphoreType.DMA,
        ],
        compiler_params=pltpu.CompilerParams(needs_layout_passes=False),
    )


_transpose = _build_transpose()
_sampler = _build_sampler()


@jax.jit
def kernel(feature_maps, sample_points):
    B, D, H, W = feature_maps.shape
    N = sample_points.shape[1]
    table = _transpose(feature_maps)
    pts = sample_points.reshape(2 * B * N)
    out = _sampler(table, pts)
    return out.reshape(B, N, D)


# speedup vs baseline: 2.0871x; 1.0637x over previous
import functools

import jax
import jax.numpy as jnp
from jax import lax
from jax.experimental import pallas as pl
from jax.experimental.pallas import tpu as pltpu
from jax.experimental.pallas import tpu_sc as plsc

L = 16
NC = 2
NS = 16
NW = NC * NS
DP = 128


def _build_transpose(batch, D=96, H=384, W=384, hb=8):
    def tbody(fm_ref, out_ref):
        x = fm_ref[0]
        x2 = x.reshape(D, hb * W)
        eye = (lax.broadcasted_iota(jnp.int32, (D, DP), 0)
               == lax.broadcasted_iota(jnp.int32, (D, DP), 1)).astype(jnp.float32)
        dn = (((0,), (0,)), ((), ()))
        out_ref[...] = lax.dot_general(x2, eye, dn,
                                       preferred_element_type=jnp.float32)

    return pl.pallas_call(
        tbody,
        grid=(H // hb,),
        in_specs=[pl.BlockSpec((1, D, hb, W), lambda h: (batch, 0, h, 0))],
        out_specs=pl.BlockSpec((hb * W, DP), lambda h: (h, 0)),
        out_shape=jax.ShapeDtypeStruct((H * W, DP), jnp.float32),
    )


def _build_sampler(batch, D=96, H=384, W=384, N=16384, chunk=64):
    per_tile = N // NW
    assert per_tile % chunk == 0
    nchunk = per_tile // chunk
    ngrp = per_tile // L
    assert chunk <= 128
    assert D % L == 0
    assert nchunk % 2 == 0
    pts_off = batch * N * 2

    def body(table, pts, out, pts_v,
             idx00, idx10, idx01, idx11,
             w00_v, w10_v, w01_v, w11_v,
             rows0, rows1, out_v, sem):
        cid = lax.axis_index("c")
        sid = lax.axis_index("s")
        wid = sid * NC + cid
        base = wid * per_tile

        pltpu.sync_copy(pts.at[pl.ds(pts_off + base * 2, per_tile * 2)], pts_v)

        lanes2 = lax.iota(jnp.int32, L) * 2

        def grp(g, carry):
            sl = pl.ds(g * L, L)
            ii = lanes2 + g * 2 * L
            x = plsc.load_gather(pts_v, [ii])
            y = plsc.load_gather(pts_v, [ii + 1])
            ix = x - 0.5
            iy = y - 0.5
            x0 = ix.astype(jnp.int32)
            x0 = jnp.where(ix < x0.astype(jnp.float32), x0 - 1, x0)
            y0 = iy.astype(jnp.int32)
            y0 = jnp.where(iy < y0.astype(jnp.float32), y0 - 1, y0)
            wx1 = ix - x0.astype(jnp.float32)
            wx0 = 1.0 - wx1
            wy1 = iy - y0.astype(jnp.float32)
            wy0 = 1.0 - wy1
            wx0 = jnp.where(x0 >= 0, wx0, 0.0)
            wx1 = jnp.where(x0 <= W - 2, wx1, 0.0)
            wy0 = jnp.where(y0 >= 0, wy0, 0.0)
            wy1 = jnp.where(y0 <= H - 2, wy1, 0.0)
            x0c = jnp.maximum(x0, 0)
            x1c = jnp.minimum(x0 + 1, W - 1)
            y0c = jnp.maximum(y0, 0)
            y1c = jnp.minimum(y0 + 1, H - 1)
            r0 = y0c * W
            r1 = y1c * W
            c = g // (chunk // L)
            o = (g % (chunk // L)) * L
            csl = pl.ds(o, L)
            idx00[c, csl] = r0 + x0c
            idx10[c, csl] = r0 + x1c
            idx01[c, csl] = r1 + x0c
            idx11[c, csl] = r1 + x1c
            w00_v[sl] = wx0 * wy0
            w10_v[sl] = wx1 * wy0
            w01_v[sl] = wx0 * wy1
            w11_v[sl] = wx1 * wy1
            return carry

        lax.fori_loop(0, ngrp, grp, 0)

        def fire(c, buf):
            pltpu.async_copy(table.at[idx00.at[c]], buf.at[0], sem)
            pltpu.async_copy(table.at[idx10.at[c]], buf.at[1], sem)
            pltpu.async_copy(table.at[idx01.at[c]], buf.at[2], sem)
            pltpu.async_copy(table.at[idx11.at[c]], buf.at[3], sem)

        def drain(c, buf):
            pltpu.make_async_copy(table.at[idx00.at[c]], buf.at[0], sem).wait()
            pltpu.make_async_copy(table.at[idx10.at[c]], buf.at[1], sem).wait()
            pltpu.make_async_copy(table.at[idx01.at[c]], buf.at[2], sem).wait()
            pltpu.make_async_copy(table.at[idx11.at[c]], buf.at[3], sem).wait()

        def combine(c, buf):
            cbase = c * chunk

            def grp16(g, carry2):
                gb = g * L
                w00v = w00_v[pl.ds(cbase + gb, L)]
                w10v = w10_v[pl.ds(cbase + gb, L)]
                w01v = w01_v[pl.ds(cbase + gb, L)]
                w11v = w11_v[pl.ds(cbase + gb, L)]
                for k in range(L):
                    p = gb + k
                    w00 = w00v[k]
                    w10 = w10v[k]
                    w01 = w01v[k]
                    w11 = w11v[k]
                    pD = p * D
                    for j in range(D // L):
                        cs = pl.ds(j * L, L)
                        acc = (w00 * buf[0, p, cs] + w10 * buf[1, p, cs]
                               + w01 * buf[2, p, cs] + w11 * buf[3, p, cs])
                        out_v[pl.ds(pD + j * L, L)] = acc
                return carry2

            lax.fori_loop(0, chunk // L, grp16, 0)
            pltpu.sync_copy(out_v, out.at[pl.ds((base + cbase) * D, chunk * D)])

        fire(0, rows0)

        def pair(i2, carry):
            c0 = i2 * 2
            drain(c0, rows0)
            fire(c0 + 1, rows1)
            combine(c0, rows0)
            drain(c0 + 1, rows1)

            @pl.when(c0 + 2 < nchunk)
            def _():
                fire(c0 + 2, rows0)

            combine(c0 + 1, rows1)
            return carry

        lax.fori_loop(0, nchunk // 2, pair, 0)

    mesh = plsc.VectorSubcoreMesh(core_axis_name="c", subcore_axis_name="s",
                                  num_cores=NC, num_subcores=NS)
    return pl.kernel(
        body,
        out_type=jax.ShapeDtypeStruct((N * D,), jnp.float32),
        mesh=mesh,
        scratch_types=[
            pltpu.VMEM((2 * per_tile,), jnp.float32),
            pltpu.VMEM((nchunk, chunk), jnp.int32),
            pltpu.VMEM((nchunk, chunk), jnp.int32),
            pltpu.VMEM((nchunk, chunk), jnp.int32),
            pltpu.VMEM((nchunk, chunk), jnp.int32),
            pltpu.VMEM((per_tile + L,), jnp.float32),
            pltpu.VMEM((per_tile + L,), jnp.float32),
            pltpu.VMEM((per_tile + L,), jnp.float32),
            pltpu.VMEM((per_tile + L,), jnp.float32),
            pltpu.VMEM((4, chunk, DP), jnp.float32),
            pltpu.VMEM((4, chunk, DP), jnp.float32),
            pltpu.VMEM((chunk * D,), jnp.float32),
            pltpu.SemaphoreType.DMA,
        ],
        compiler_params=pltpu.CompilerParams(needs_layout_passes=False),
    )


_transpose_b = [_build_transpose(0), _build_transpose(1)]
_sampler_b = [_build_sampler(0), _build_sampler(1)]


@jax.jit
def kernel(feature_maps, sample_points):
    B, D, H, W = feature_maps.shape
    N = sample_points.shape[1]
    pts = sample_points.reshape(2 * B * N)
    outs = []
    for b in range(B):
        table = _transpose_b[b](feature_maps)
        outs.append(_sampler_b[b](table, pts).reshape(N, D))
    return jnp.stack(outs)


# speedup vs baseline: 2.3289x; 1.1158x over previous
import functools

import jax
import jax.numpy as jnp
from jax import lax
from jax.experimental import pallas as pl
from jax.experimental.pallas import tpu as pltpu
from jax.experimental.pallas import tpu_sc as plsc

L = 16
NC = 2
NS = 16
NW = NC * NS
DP = 128


def _build_transpose(batch, D=96, H=384, W=384, hb=16):
    def tbody(fm_ref, out_ref):
        x = fm_ref[0]
        x2 = x.reshape(D, hb * W)
        eye = (lax.broadcasted_iota(jnp.int32, (D, DP), 0)
               == lax.broadcasted_iota(jnp.int32, (D, DP), 1)).astype(jnp.float32)
        dn = (((0,), (0,)), ((), ()))
        out_ref[...] = lax.dot_general(x2, eye, dn,
                                       preferred_element_type=jnp.float32)

    return pl.pallas_call(
        tbody,
        grid=(H // hb,),
        in_specs=[pl.BlockSpec((1, D, hb, W), lambda h: (batch, 0, h, 0))],
        out_specs=pl.BlockSpec((hb * W, DP), lambda h: (h, 0)),
        out_shape=jax.ShapeDtypeStruct((H * W, DP), jnp.float32),
    )


def _build_sampler(batch, D=96, H=384, W=384, N=16384, chunk=64):
    per_tile = N // NW
    assert per_tile % chunk == 0
    nchunk = per_tile // chunk
    ngrp = per_tile // L
    assert chunk <= 128
    assert D % L == 0
    assert nchunk % 2 == 0
    pts_off = batch * N * 2

    def body(table, pts, out, pts_v,
             idx00, idx10, idx01, idx11,
             w00_v, w10_v, w01_v, w11_v,
             rows0, rows1, out_v, sem):
        cid = lax.axis_index("c")
        sid = lax.axis_index("s")
        wid = sid * NC + cid
        base = wid * per_tile

        pltpu.sync_copy(pts.at[pl.ds(pts_off + base * 2, per_tile * 2)], pts_v)

        lanes2 = lax.iota(jnp.int32, L) * 2

        def grp(g, carry):
            sl = pl.ds(g * L, L)
            ii = lanes2 + g * 2 * L
            x = plsc.load_gather(pts_v, [ii])
            y = plsc.load_gather(pts_v, [ii + 1])
            ix = x - 0.5
            iy = y - 0.5
            x0 = ix.astype(jnp.int32)
            x0 = jnp.where(ix < x0.astype(jnp.float32), x0 - 1, x0)
            y0 = iy.astype(jnp.int32)
            y0 = jnp.where(iy < y0.astype(jnp.float32), y0 - 1, y0)
            wx1 = ix - x0.astype(jnp.float32)
            wx0 = 1.0 - wx1
            wy1 = iy - y0.astype(jnp.float32)
            wy0 = 1.0 - wy1
            wx0 = jnp.where(x0 >= 0, wx0, 0.0)
            wx1 = jnp.where(x0 <= W - 2, wx1, 0.0)
            wy0 = jnp.where(y0 >= 0, wy0, 0.0)
            wy1 = jnp.where(y0 <= H - 2, wy1, 0.0)
            x0c = jnp.maximum(x0, 0)
            x1c = jnp.minimum(x0 + 1, W - 1)
            y0c = jnp.maximum(y0, 0)
            y1c = jnp.minimum(y0 + 1, H - 1)
            r0 = y0c * W
            r1 = y1c * W
            c = g // (chunk // L)
            o = (g % (chunk // L)) * L
            csl = pl.ds(o, L)
            idx00[c, csl] = r0 + x0c
            idx10[c, csl] = r0 + x1c
            idx01[c, csl] = r1 + x0c
            idx11[c, csl] = r1 + x1c
            w00_v[sl] = wx0 * wy0
            w10_v[sl] = wx1 * wy0
            w01_v[sl] = wx0 * wy1
            w11_v[sl] = wx1 * wy1
            return carry

        lax.fori_loop(0, ngrp, grp, 0)

        def fire(c, buf):
            pltpu.async_copy(table.at[idx00.at[c]], buf.at[0], sem)
            pltpu.async_copy(table.at[idx10.at[c]], buf.at[1], sem)
            pltpu.async_copy(table.at[idx01.at[c]], buf.at[2], sem)
            pltpu.async_copy(table.at[idx11.at[c]], buf.at[3], sem)

        def drain(c, buf):
            pltpu.make_async_copy(table.at[idx00.at[c]], buf.at[0], sem).wait()
            pltpu.make_async_copy(table.at[idx10.at[c]], buf.at[1], sem).wait()
            pltpu.make_async_copy(table.at[idx01.at[c]], buf.at[2], sem).wait()
            pltpu.make_async_copy(table.at[idx11.at[c]], buf.at[3], sem).wait()

        def combine(c, buf):
            cbase = c * chunk

            def grp16(g, carry2):
                gb = g * L
                w00v = w00_v[pl.ds(cbase + gb, L)]
                w10v = w10_v[pl.ds(cbase + gb, L)]
                w01v = w01_v[pl.ds(cbase + gb, L)]
                w11v = w11_v[pl.ds(cbase + gb, L)]
                for k in range(L):
                    p = gb + k
                    w00 = w00v[k]
                    w10 = w10v[k]
                    w01 = w01v[k]
                    w11 = w11v[k]
                    pD = p * D
                    for j in range(D // L):
                        cs = pl.ds(j * L, L)
                        acc = (w00 * buf[0, p, cs] + w10 * buf[1, p, cs]
                               + w01 * buf[2, p, cs] + w11 * buf[3, p, cs])
                        out_v[pl.ds(pD + j * L, L)] = acc
                return carry2

            lax.fori_loop(0, chunk // L, grp16, 0)
            pltpu.sync_copy(out_v, out.at[pl.ds((base + cbase) * D, chunk * D)])

        fire(0, rows0)

        def pair(i2, carry):
            c0 = i2 * 2
            drain(c0, rows0)
            fire(c0 + 1, rows1)
            combine(c0, rows0)
            drain(c0 + 1, rows1)

            @pl.when(c0 + 2 < nchunk)
            def _():
                fire(c0 + 2, rows0)

            combine(c0 + 1, rows1)
            return carry

        lax.fori_loop(0, nchunk // 2, pair, 0)

    mesh = plsc.VectorSubcoreMesh(core_axis_name="c", subcore_axis_name="s",
                                  num_cores=NC, num_subcores=NS)
    return pl.kernel(
        body,
        out_type=jax.ShapeDtypeStruct((N * D,), jnp.float32),
        mesh=mesh,
        scratch_types=[
            pltpu.VMEM((2 * per_tile,), jnp.float32),
            pltpu.VMEM((nchunk, chunk), jnp.int32),
            pltpu.VMEM((nchunk, chunk), jnp.int32),
            pltpu.VMEM((nchunk, chunk), jnp.int32),
            pltpu.VMEM((nchunk, chunk), jnp.int32),
            pltpu.VMEM((per_tile + L,), jnp.float32),
            pltpu.VMEM((per_tile + L,), jnp.float32),
            pltpu.VMEM((per_tile + L,), jnp.float32),
            pltpu.VMEM((per_tile + L,), jnp.float32),
            pltpu.VMEM((4, chunk, DP), jnp.float32),
            pltpu.VMEM((4, chunk, DP), jnp.float32),
            pltpu.VMEM((chunk * D,), jnp.float32),
            pltpu.SemaphoreType.DMA,
        ],
        compiler_params=pltpu.CompilerParams(needs_layout_passes=False),
    )


_transpose_b = [_build_transpose(0), _build_transpose(1)]
_sampler_b = [_build_sampler(0), _build_sampler(1)]


@jax.jit
def kernel(feature_maps, sample_points):
    B, D, H, W = feature_maps.shape
    N = sample_points.shape[1]
    pts = sample_points.reshape(2 * B * N)
    outs = []
    for b in range(B):
        table = _transpose_b[b](feature_maps)
        outs.append(_sampler_b[b](table, pts).reshape(N, D))
    return jnp.stack(outs)


# speedup vs baseline: 2.4071x; 1.0336x over previous
import functools

import jax
import jax.numpy as jnp
from jax import lax
from jax.experimental import pallas as pl
from jax.experimental.pallas import tpu as pltpu
from jax.experimental.pallas import tpu_sc as plsc

L = 16
NC = 2
NS = 16
NW = NC * NS
DP = 128


def _build_transpose(batch, D=96, H=384, W=384, hb=32):
    def tbody(fm_ref, out_ref):
        x = fm_ref[0]
        x2 = x.reshape(D, hb * W)
        eye = (lax.broadcasted_iota(jnp.int32, (D, DP), 0)
               == lax.broadcasted_iota(jnp.int32, (D, DP), 1)).astype(jnp.float32)
        dn = (((0,), (0,)), ((), ()))
        out_ref[...] = lax.dot_general(x2, eye, dn,
                                       preferred_element_type=jnp.float32)

    return pl.pallas_call(
        tbody,
        grid=(H // hb,),
        in_specs=[pl.BlockSpec((1, D, hb, W), lambda h: (batch, 0, h, 0))],
        out_specs=pl.BlockSpec((hb * W, DP), lambda h: (h, 0)),
        out_shape=jax.ShapeDtypeStruct((H * W, DP), jnp.float32),
    )


def _build_sampler(batch, D=96, H=384, W=384, N=16384, chunk=64):
    per_tile = N // NW
    assert per_tile % chunk == 0
    nchunk = per_tile // chunk
    ngrp = per_tile // L
    assert chunk <= 128
    assert D % L == 0
    assert nchunk % 2 == 0
    pts_off = batch * N * 2

    def body(table, pts, out, pts_v,
             idx00, idx10, idx01, idx11,
             w00_v, w10_v, w01_v, w11_v,
             rows0, rows1, out_v, sem):
        cid = lax.axis_index("c")
        sid = lax.axis_index("s")
        wid = sid * NC + cid
        base = wid * per_tile

        pltpu.sync_copy(pts.at[pl.ds(pts_off + base * 2, per_tile * 2)], pts_v)

        lanes2 = lax.iota(jnp.int32, L) * 2

        def grp(g, carry):
            sl = pl.ds(g * L, L)
            ii = lanes2 + g * 2 * L
            x = plsc.load_gather(pts_v, [ii])
            y = plsc.load_gather(pts_v, [ii + 1])
            ix = x - 0.5
            iy = y - 0.5
            x0 = ix.astype(jnp.int32)
            x0 = jnp.where(ix < x0.astype(jnp.float32), x0 - 1, x0)
            y0 = iy.astype(jnp.int32)
            y0 = jnp.where(iy < y0.astype(jnp.float32), y0 - 1, y0)
            wx1 = ix - x0.astype(jnp.float32)
            wx0 = 1.0 - wx1
            wy1 = iy - y0.astype(jnp.float32)
            wy0 = 1.0 - wy1
            wx0 = jnp.where(x0 >= 0, wx0, 0.0)
            wx1 = jnp.where(x0 <= W - 2, wx1, 0.0)
            wy0 = jnp.where(y0 >= 0, wy0, 0.0)
            wy1 = jnp.where(y0 <= H - 2, wy1, 0.0)
            x0c = jnp.maximum(x0, 0)
            x1c = jnp.minimum(x0 + 1, W - 1)
            y0c = jnp.maximum(y0, 0)
            y1c = jnp.minimum(y0 + 1, H - 1)
            r0 = y0c * W
            r1 = y1c * W
            c = g // (chunk // L)
            o = (g % (chunk // L)) * L
            csl = pl.ds(o, L)
            idx00[c, csl] = r0 + x0c
            idx10[c, csl] = r0 + x1c
            idx01[c, csl] = r1 + x0c
            idx11[c, csl] = r1 + x1c
            w00_v[sl] = wx0 * wy0
            w10_v[sl] = wx1 * wy0
            w01_v[sl] = wx0 * wy1
            w11_v[sl] = wx1 * wy1
            return carry

        lax.fori_loop(0, ngrp, grp, 0)

        def fire(c, buf):
            pltpu.async_copy(table.at[idx00.at[c]], buf.at[0], sem)
            pltpu.async_copy(table.at[idx10.at[c]], buf.at[1], sem)
            pltpu.async_copy(table.at[idx01.at[c]], buf.at[2], sem)
            pltpu.async_copy(table.at[idx11.at[c]], buf.at[3], sem)

        def drain(c, buf):
            pltpu.make_async_copy(table.at[idx00.at[c]], buf.at[0], sem).wait()
            pltpu.make_async_copy(table.at[idx10.at[c]], buf.at[1], sem).wait()
            pltpu.make_async_copy(table.at[idx01.at[c]], buf.at[2], sem).wait()
            pltpu.make_async_copy(table.at[idx11.at[c]], buf.at[3], sem).wait()

        def combine(c, buf):
            cbase = c * chunk

            def grp16(g, carry2):
                gb = g * L
                w00v = w00_v[pl.ds(cbase + gb, L)]
                w10v = w10_v[pl.ds(cbase + gb, L)]
                w01v = w01_v[pl.ds(cbase + gb, L)]
                w11v = w11_v[pl.ds(cbase + gb, L)]
                for k in range(L):
                    p = gb + k
                    w00 = w00v[k]
                    w10 = w10v[k]
                    w01 = w01v[k]
                    w11 = w11v[k]
                    pD = p * D
                    for j in range(D // L):
                        cs = pl.ds(j * L, L)
                        acc = (w00 * buf[0, p, cs] + w10 * buf[1, p, cs]
                               + w01 * buf[2, p, cs] + w11 * buf[3, p, cs])
                        out_v[pl.ds(pD + j * L, L)] = acc
                return carry2

            lax.fori_loop(0, chunk // L, grp16, 0)
            pltpu.sync_copy(out_v, out.at[pl.ds((base + cbase) * D, chunk * D)])

        fire(0, rows0)

        def pair(i2, carry):
            c0 = i2 * 2
            drain(c0, rows0)
            fire(c0 + 1, rows1)
            combine(c0, rows0)
            drain(c0 + 1, rows1)

            @pl.when(c0 + 2 < nchunk)
            def _():
                fire(c0 + 2, rows0)

            combine(c0 + 1, rows1)
            return carry

        lax.fori_loop(0, nchunk // 2, pair, 0)

    mesh = plsc.VectorSubcoreMesh(core_axis_name="c", subcore_axis_name="s",
                                  num_cores=NC, num_subcores=NS)
    return pl.kernel(
        body,
        out_type=jax.ShapeDtypeStruct((N * D,), jnp.float32),
        mesh=mesh,
        scratch_types=[
            pltpu.VMEM((2 * per_tile,), jnp.float32),
            pltpu.VMEM((nchunk, chunk), jnp.int32),
            pltpu.VMEM((nchunk, chunk), jnp.int32),
            pltpu.VMEM((nchunk, chunk), jnp.int32),
            pltpu.VMEM((nchunk, chunk), jnp.int32),
            pltpu.VMEM((per_tile + L,), jnp.float32),
            pltpu.VMEM((per_tile + L,), jnp.float32),
            pltpu.VMEM((per_tile + L,), jnp.float32),
            pltpu.VMEM((per_tile + L,), jnp.float32),
            pltpu.VMEM((4, chunk, DP), jnp.float32),
            pltpu.VMEM((4, chunk, DP), jnp.float32),
            pltpu.VMEM((chunk * D,), jnp.float32),
            pltpu.SemaphoreType.DMA,
        ],
        compiler_params=pltpu.CompilerParams(needs_layout_passes=False),
    )


_transpose_b = [_build_transpose(0), _build_transpose(1)]
_sampler_b = [_build_sampler(0), _build_sampler(1)]


@jax.jit
def kernel(feature_maps, sample_points):
    B, D, H, W = feature_maps.shape
    N = sample_points.shape[1]
    pts = sample_points.reshape(2 * B * N)
    outs = []
    for b in range(B):
        table = _transpose_b[b](feature_maps)
        outs.append(_sampler_b[b](table, pts).reshape(N, D))
    return jnp.stack(outs)


# speedup vs baseline: 2.4289x; 1.0091x over previous
import functools

import jax
import jax.numpy as jnp
from jax import lax
from jax.experimental import pallas as pl
from jax.experimental.pallas import tpu as pltpu
from jax.experimental.pallas import tpu_sc as plsc

L = 16
NC = 2
NS = 16
NW = NC * NS
DP = 128


def _build_transpose(batch, D=96, H=384, W=384, hb=48):
    def tbody(fm_ref, out_ref):
        x = fm_ref[0]
        x2 = x.reshape(D, hb * W)
        eye = (lax.broadcasted_iota(jnp.int32, (D, DP), 0)
               == lax.broadcasted_iota(jnp.int32, (D, DP), 1)).astype(jnp.float32)
        dn = (((0,), (0,)), ((), ()))
        out_ref[...] = lax.dot_general(x2, eye, dn,
                                       preferred_element_type=jnp.float32)

    return pl.pallas_call(
        tbody,
        grid=(H // hb,),
        in_specs=[pl.BlockSpec((1, D, hb, W), lambda h: (batch, 0, h, 0))],
        out_specs=pl.BlockSpec((hb * W, DP), lambda h: (h, 0)),
        out_shape=jax.ShapeDtypeStruct((H * W, DP), jnp.float32),
    )


def _build_sampler(batch, D=96, H=384, W=384, N=16384, chunk=64):
    per_tile = N // NW
    assert per_tile % chunk == 0
    nchunk = per_tile // chunk
    ngrp = per_tile // L
    assert chunk <= 128
    assert D % L == 0
    assert nchunk % 2 == 0
    pts_off = batch * N * 2

    def body(table, pts, out, pts_v,
             idx00, idx10, idx01, idx11,
             w00_v, w10_v, w01_v, w11_v,
             rows0, rows1, out_v, sem):
        cid = lax.axis_index("c")
        sid = lax.axis_index("s")
        wid = sid * NC + cid
        base = wid * per_tile

        pltpu.sync_copy(pts.at[pl.ds(pts_off + base * 2, per_tile * 2)], pts_v)

        lanes2 = lax.iota(jnp.int32, L) * 2

        def grp(g, carry):
            sl = pl.ds(g * L, L)
            ii = lanes2 + g * 2 * L
            x = plsc.load_gather(pts_v, [ii])
            y = plsc.load_gather(pts_v, [ii + 1])
            ix = x - 0.5
            iy = y - 0.5
            x0 = ix.astype(jnp.int32)
            x0 = jnp.where(ix < x0.astype(jnp.float32), x0 - 1, x0)
            y0 = iy.astype(jnp.int32)
            y0 = jnp.where(iy < y0.astype(jnp.float32), y0 - 1, y0)
            wx1 = ix - x0.astype(jnp.float32)
            wx0 = 1.0 - wx1
            wy1 = iy - y0.astype(jnp.float32)
            wy0 = 1.0 - wy1
            wx0 = jnp.where(x0 >= 0, wx0, 0.0)
            wx1 = jnp.where(x0 <= W - 2, wx1, 0.0)
            wy0 = jnp.where(y0 >= 0, wy0, 0.0)
            wy1 = jnp.where(y0 <= H - 2, wy1, 0.0)
            x0c = jnp.maximum(x0, 0)
            x1c = jnp.minimum(x0 + 1, W - 1)
            y0c = jnp.maximum(y0, 0)
            y1c = jnp.minimum(y0 + 1, H - 1)
            r0 = y0c * W
            r1 = y1c * W
            c = g // (chunk // L)
            o = (g % (chunk // L)) * L
            csl = pl.ds(o, L)
            idx00[c, csl] = r0 + x0c
            idx10[c, csl] = r0 + x1c
            idx01[c, csl] = r1 + x0c
            idx11[c, csl] = r1 + x1c
            w00_v[sl] = wx0 * wy0
            w10_v[sl] = wx1 * wy0
            w01_v[sl] = wx0 * wy1
            w11_v[sl] = wx1 * wy1
            return carry

        lax.fori_loop(0, ngrp, grp, 0)

        def fire(c, buf):
            pltpu.async_copy(table.at[idx00.at[c]], buf.at[0], sem)
            pltpu.async_copy(table.at[idx10.at[c]], buf.at[1], sem)
            pltpu.async_copy(table.at[idx01.at[c]], buf.at[2], sem)
            pltpu.async_copy(table.at[idx11.at[c]], buf.at[3], sem)

        def drain(c, buf):
            pltpu.make_async_copy(table.at[idx00.at[c]], buf.at[0], sem).wait()
            pltpu.make_async_copy(table.at[idx10.at[c]], buf.at[1], sem).wait()
            pltpu.make_async_copy(table.at[idx01.at[c]], buf.at[2], sem).wait()
            pltpu.make_async_copy(table.at[idx11.at[c]], buf.at[3], sem).wait()

        def combine(c, buf):
            cbase = c * chunk

            def grp16(g, carry2):
                gb = g * L
                w00v = w00_v[pl.ds(cbase + gb, L)]
                w10v = w10_v[pl.ds(cbase + gb, L)]
                w01v = w01_v[pl.ds(cbase + gb, L)]
                w11v = w11_v[pl.ds(cbase + gb, L)]
                for k in range(L):
                    p = gb + k
                    w00 = w00v[k]
                    w10 = w10v[k]
                    w01 = w01v[k]
                    w11 = w11v[k]
                    pD = p * D
                    for j in range(D // L):
                        cs = pl.ds(j * L, L)
                        acc = (w00 * buf[0, p, cs] + w10 * buf[1, p, cs]
                               + w01 * buf[2, p, cs] + w11 * buf[3, p, cs])
                        out_v[pl.ds(pD + j * L, L)] = acc
                return carry2

            lax.fori_loop(0, chunk // L, grp16, 0)
            pltpu.sync_copy(out_v, out.at[pl.ds((base + cbase) * D, chunk * D)])

        fire(0, rows0)

        def pair(i2, carry):
            c0 = i2 * 2
            drain(c0, rows0)
            fire(c0 + 1, rows1)
            combine(c0, rows0)
            drain(c0 + 1, rows1)

            @pl.when(c0 + 2 < nchunk)
            def _():
                fire(c0 + 2, rows0)

            combine(c0 + 1, rows1)
            return carry

        lax.fori_loop(0, nchunk // 2, pair, 0)

    mesh = plsc.VectorSubcoreMesh(core_axis_name="c", subcore_axis_name="s",
                                  num_cores=NC, num_subcores=NS)
    return pl.kernel(
        body,
        out_type=jax.ShapeDtypeStruct((N * D,), jnp.float32),
        mesh=mesh,
        scratch_types=[
            pltpu.VMEM((2 * per_tile,), jnp.float32),
            pltpu.VMEM((nchunk, chunk), jnp.int32),
            pltpu.VMEM((nchunk, chunk), jnp.int32),
            pltpu.VMEM((nchunk, chunk), jnp.int32),
            pltpu.VMEM((nchunk, chunk), jnp.int32),
            pltpu.VMEM((per_tile + L,), jnp.float32),
            pltpu.VMEM((per_tile + L,), jnp.float32),
            pltpu.VMEM((per_tile + L,), jnp.float32),
            pltpu.VMEM((per_tile + L,), jnp.float32),
            pltpu.VMEM((4, chunk, DP), jnp.float32),
            pltpu.VMEM((4, chunk, DP), jnp.float32),
            pltpu.VMEM((chunk * D,), jnp.float32),
            pltpu.SemaphoreType.DMA,
        ],
        compiler_params=pltpu.CompilerParams(needs_layout_passes=False),
    )


_transpose_b = [_build_transpose(0), _build_transpose(1)]
_sampler_b = [_build_sampler(0), _build_sampler(1)]


@jax.jit
def kernel(feature_maps, sample_points):
    B, D, H, W = feature_maps.shape
    N = sample_points.shape[1]
    pts = sample_points.reshape(2 * B * N)
    outs = []
    for b in range(B):
        table = _transpose_b[b](feature_maps)
        outs.append(_sampler_b[b](table, pts).reshape(N, D))
    return jnp.stack(outs)
